# baseline (speedup 1.0000x reference)
import functools
import math

import jax
import jax.numpy as jnp
from jax.experimental import pallas as pl

H = 16
D = 64
HD = H * D
R = 4
EPS = 1e-6
F32 = jnp.float32
BF16 = jnp.bfloat16
HIGHEST = jax.lax.Precision.HIGHEST


def _head_pool():
    i = jax.lax.broadcasted_iota(jnp.int32, (HD, H), 0) // D
    j = jax.lax.broadcasted_iota(jnp.int32, (HD, H), 1)
    return (i == j).astype(F32)


def _head_bcast():
    i = jax.lax.broadcasted_iota(jnp.int32, (H, HD), 0)
    j = jax.lax.broadcasted_iota(jnp.int32, (H, HD), 1) // D
    return (i == j).astype(F32)


def _rms_scale(v):
    ssq = jnp.dot(v * v, _head_pool(), preferred_element_type=F32, precision=HIGHEST)
    ssqb = jnp.dot(ssq, _head_bcast(), preferred_element_type=F32, precision=HIGHEST)
    return jax.lax.rsqrt(ssqb * (1.0 / D) + EPS)


def _keys_kernel(x_ref, wk_ref, wg_ref, ape_ref, out_ref, *, e_dim):
    x = x_ref[0].astype(BF16)
    ks, gs = [], []
    for r in range(R):
        xr = x[:, r * e_dim:(r + 1) * e_dim]
        ks.append(jnp.dot(xr, wk_ref[...], preferred_element_type=F32))
        gs.append(jnp.dot(xr, wg_ref[...], preferred_element_type=F32)
                  + ape_ref[r, :][None, :])
    m = jnp.maximum(jnp.maximum(gs[0], gs[1]), jnp.maximum(gs[2], gs[3]))
    es = [jnp.exp(gr - m) for gr in gs]
    denom = ((es[0] + es[1]) + es[2]) + es[3]
    ws = [er / denom for er in es]
    keys = ((ks[0] * ws[0] + ks[1] * ws[1]) + ks[2] * ws[2]) + ks[3] * ws[3]
    out_ref[0] = (keys * _rms_scale(keys)).astype(BF16)


def _scores_kernel(x_ref, wq_ref, keys_ref, out_ref, *, bt, g_tot, topk):
    tb = pl.program_id(1)
    x = x_ref[0].astype(BF16)
    q = jnp.dot(x, wq_ref[...], preferred_element_type=F32)
    qn = (q * _rms_scale(q)).astype(BF16)
    s = jax.lax.dot_general(qn, keys_ref[0], (((1,), (1,)), ((), ())),
                            preferred_element_type=F32)
    t_idx = tb * bt + jax.lax.broadcasted_iota(jnp.int32, (bt, g_tot), 0)
    g_end = R * jax.lax.broadcasted_iota(jnp.int32, (bt, g_tot), 1) + (R - 1)
    causal = g_end <= t_idx
    neg = jnp.float32(-jnp.inf)
    s = jnp.where(causal, s, neg)
    cur = s
    th = None
    for _ in range(topk):
        th = jnp.max(cur, axis=1, keepdims=True)
        cur = jnp.where(cur >= th, neg, cur)
    out_ref[0] = causal & (s >= th)


def kernel(x, Wq, Wk, Wg, ape):
    B, T, E = x.shape
    G = T // R
    BG = min(128, G)
    BT = min(256, T)
    wq = Wq.astype(BF16)
    wk = Wk.astype(BF16)
    wg = Wg.astype(BF16)
    ape2 = ape.reshape(R, HD)
    x4 = x.reshape(B, G, R * E)
    keys = pl.pallas_call(
        functools.partial(_keys_kernel, e_dim=E),
        grid=(B, G // BG),
        in_specs=[
            pl.BlockSpec((1, BG, R * E), lambda b, gb: (b, gb, 0)),
            pl.BlockSpec((E, HD), lambda b, gb: (0, 0)),
            pl.BlockSpec((E, HD), lambda b, gb: (0, 0)),
            pl.BlockSpec((R, HD), lambda b, gb: (0, 0)),
        ],
        out_specs=pl.BlockSpec((1, BG, HD), lambda b, gb: (b, gb, 0)),
        out_shape=jax.ShapeDtypeStruct((B, G, HD), BF16),
    )(x4, wk, wg, ape2)
    topk = min(8, G)
    mask = pl.pallas_call(
        functools.partial(_scores_kernel, bt=BT, g_tot=G, topk=topk),
        grid=(B, T // BT),
        in_specs=[
            pl.BlockSpec((1, BT, E), lambda b, tb: (b, tb, 0)),
            pl.BlockSpec((E, HD), lambda b, tb: (0, 0)),
            pl.BlockSpec((1, G, HD), lambda b, tb: (b, 0, 0)),
        ],
        out_specs=pl.BlockSpec((1, BT, G), lambda b, tb: (b, tb, 0)),
        out_shape=jax.ShapeDtypeStruct((B, T, G), jnp.bool_),
    )(x, wq, keys)
    group_ends = jnp.minimum(jnp.arange(R - 1, G * R, R), T - 1)
    return mask, group_ends


# speedup vs baseline: 1.0148x; 1.0148x over previous
import functools
import math

import jax
import jax.numpy as jnp
from jax.experimental import pallas as pl

H = 16
D = 64
HD = H * D
R = 4
EPS = 1e-6
F32 = jnp.float32
BF16 = jnp.bfloat16
HIGHEST = jax.lax.Precision.HIGHEST


def _head_pool():
    i = jax.lax.broadcasted_iota(jnp.int32, (HD, H), 0) // D
    j = jax.lax.broadcasted_iota(jnp.int32, (HD, H), 1)
    return (i == j).astype(F32)


def _head_bcast():
    i = jax.lax.broadcasted_iota(jnp.int32, (H, HD), 0)
    j = jax.lax.broadcasted_iota(jnp.int32, (H, HD), 1) // D
    return (i == j).astype(F32)


def _rms_scale(v):
    ssq = jnp.dot(v * v, _head_pool(), preferred_element_type=F32,
                  precision=HIGHEST)
    ssqb = jnp.dot(ssq, _head_bcast(), preferred_element_type=F32,
                   precision=HIGHEST)
    return jax.lax.rsqrt(ssqb * (1.0 / D) + EPS)


def _keys_kernel(x_ref, wk_ref, wg_ref, ape_ref, out_ref, *, e_dim):
    x = x_ref[0].astype(BF16)
    ks, gs = [], []
    for r in range(R):
        xr = x[:, r * e_dim:(r + 1) * e_dim]
        ks.append(jnp.dot(xr, wk_ref[...], preferred_element_type=F32))
        gs.append(jnp.dot(xr, wg_ref[...], preferred_element_type=F32)
                  + ape_ref[r, :][None, :])
    m = jnp.maximum(jnp.maximum(gs[0], gs[1]), jnp.maximum(gs[2], gs[3]))
    es = [jnp.exp(gr - m) for gr in gs]
    denom = ((es[0] + es[1]) + es[2]) + es[3]
    ws = [er / denom for er in es]
    keys = ((ks[0] * ws[0] + ks[1] * ws[1]) + ks[2] * ws[2]) + ks[3] * ws[3]
    out_ref[0] = (keys * _rms_scale(keys)).astype(BF16)


def _scores_kernel(x_ref, wq_ref, keys_ref, out_ref, *, bt, g_tot, topk):
    tb = pl.program_id(1)
    x = x_ref[0].astype(BF16)
    q = jnp.dot(x, wq_ref[...], preferred_element_type=F32)
    qn = (q * _rms_scale(q)).astype(BF16)
    s = jax.lax.dot_general(qn, keys_ref[0], (((1,), (1,)), ((), ())),
                            preferred_element_type=F32)
    t_idx = tb * bt + jax.lax.broadcasted_iota(jnp.int32, (bt, g_tot), 0)
    g_end = R * jax.lax.broadcasted_iota(jnp.int32, (bt, g_tot), 1) + (R - 1)
    causal = g_end <= t_idx
    neg = jnp.float32(-jnp.inf)
    s = jnp.where(causal, s, neg)
    cur = s
    th = None
    for _ in range(topk):
        th = jnp.max(cur, axis=1, keepdims=True)
        cur = jnp.where(cur >= th, neg, cur)
    out_ref[0] = causal & (s >= th)


def kernel(x, Wq, Wk, Wg, ape):
    B, T, E = x.shape
    G = T // R
    BG = min(256, G)
    BT = min(512, T)
    wq = Wq.astype(BF16)
    wk = Wk.astype(BF16)
    wg = Wg.astype(BF16)
    ape2 = ape.reshape(R, HD)
    x4 = x.reshape(B, G, R * E)
    keys = pl.pallas_call(
        functools.partial(_keys_kernel, e_dim=E),
        grid=(B, G // BG),
        in_specs=[
            pl.BlockSpec((1, BG, R * E), lambda b, gb: (b, gb, 0)),
            pl.BlockSpec((E, HD), lambda b, gb: (0, 0)),
            pl.BlockSpec((E, HD), lambda b, gb: (0, 0)),
            pl.BlockSpec((R, HD), lambda b, gb: (0, 0)),
        ],
        out_specs=pl.BlockSpec((1, BG, HD), lambda b, gb: (b, gb, 0)),
        out_shape=jax.ShapeDtypeStruct((B, G, HD), BF16),
    )(x4, wk, wg, ape2)
    topk = min(8, G)
    mask = pl.pallas_call(
        functools.partial(_scores_kernel, bt=BT, g_tot=G, topk=topk),
        grid=(B, T // BT),
        in_specs=[
            pl.BlockSpec((1, BT, E), lambda b, tb: (b, tb, 0)),
            pl.BlockSpec((E, HD), lambda b, tb: (0, 0)),
            pl.BlockSpec((1, G, HD), lambda b, tb: (b, 0, 0)),
        ],
        out_specs=pl.BlockSpec((1, BT, G), lambda b, tb: (b, tb, 0)),
        out_shape=jax.ShapeDtypeStruct((B, T, G), jnp.bool_),
    )(x, wq, keys)
    group_ends = jnp.minimum(jnp.arange(R - 1, G * R, R), T - 1)
    return mask, group_ends


# speedup vs baseline: 1.0506x; 1.0353x over previous
import functools
import math

import jax
import jax.numpy as jnp
from jax.experimental import pallas as pl

H = 16
D = 64
HD = H * D
R = 4
EPS = 1e-6
F32 = jnp.float32
BF16 = jnp.bfloat16
HIGHEST = jax.lax.Precision.HIGHEST


def _head_pool():
    i = jax.lax.broadcasted_iota(jnp.int32, (HD, H), 0) // D
    j = jax.lax.broadcasted_iota(jnp.int32, (HD, H), 1)
    return (i == j).astype(F32)


def _head_bcast():
    i = jax.lax.broadcasted_iota(jnp.int32, (H, HD), 0)
    j = jax.lax.broadcasted_iota(jnp.int32, (H, HD), 1) // D
    return (i == j).astype(F32)


def _rms_scale(v):
    ssq = jnp.dot(v * v, _head_pool(), preferred_element_type=F32,
                  precision=HIGHEST)
    ssqb = jnp.dot(ssq, _head_bcast(), preferred_element_type=F32,
                   precision=HIGHEST)
    return jax.lax.rsqrt(ssqb * (1.0 / D) + EPS)


def _keys_kernel(x_ref, wk_ref, wg_ref, ape_ref, out_ref, *, bg):
    x = x_ref[0].astype(BF16)
    k = jnp.dot(x, wk_ref[...], preferred_element_type=F32)
    g = jnp.dot(x, wg_ref[...], preferred_element_type=F32)
    g = g + ape_ref[...]
    g = g - jnp.max(g)
    e = jnp.exp(g)
    rows = R * bg
    seg = (jax.lax.broadcasted_iota(jnp.int32, (bg, rows), 0)
           == jax.lax.broadcasted_iota(jnp.int32, (bg, rows), 1) // R)
    segf = seg.astype(F32)
    denom = jnp.dot(segf, e, preferred_element_type=F32, precision=HIGHEST)
    num = jnp.dot(segf, e * k, preferred_element_type=F32, precision=HIGHEST)
    keys = num / denom
    out_ref[0] = (keys * _rms_scale(keys)).astype(BF16)


def _scores_kernel(x_ref, wq_ref, keys_ref, out_ref, *, bt, g_tot, topk):
    tb = pl.program_id(1)
    x = x_ref[0].astype(BF16)
    q = jnp.dot(x, wq_ref[...], preferred_element_type=F32)
    qn = (q * _rms_scale(q)).astype(BF16)
    s = jax.lax.dot_general(qn, keys_ref[0], (((1,), (1,)), ((), ())),
                            preferred_element_type=F32)
    t_idx = tb * bt + jax.lax.broadcasted_iota(jnp.int32, (bt, g_tot), 0)
    g_end = R * jax.lax.broadcasted_iota(jnp.int32, (bt, g_tot), 1) + (R - 1)
    causal = g_end <= t_idx
    neg = jnp.float32(-jnp.inf)
    s = jnp.where(causal, s, neg)
    cur = s
    th = None
    for _ in range(topk):
        th = jnp.max(cur, axis=1, keepdims=True)
        cur = jnp.where(cur >= th, neg, cur)
    out_ref[0] = causal & (s >= th)


def kernel(x, Wq, Wk, Wg, ape):
    B, T, E = x.shape
    G = T // R
    BG = min(256, G)
    BT = min(512, T)
    wq = Wq.astype(BF16)
    wk = Wk.astype(BF16)
    wg = Wg.astype(BF16)
    ape2 = jnp.tile(ape.reshape(R, HD), (BG, 1))
    keys = pl.pallas_call(
        functools.partial(_keys_kernel, bg=BG),
        grid=(B, G // BG),
        in_specs=[
            pl.BlockSpec((1, R * BG, E), lambda b, gb: (b, gb, 0)),
            pl.BlockSpec((E, HD), lambda b, gb: (0, 0)),
            pl.BlockSpec((E, HD), lambda b, gb: (0, 0)),
            pl.BlockSpec((R * BG, HD), lambda b, gb: (0, 0)),
        ],
        out_specs=pl.BlockSpec((1, BG, HD), lambda b, gb: (b, gb, 0)),
        out_shape=jax.ShapeDtypeStruct((B, G, HD), BF16),
    )(x, wk, wg, ape2)
    topk = min(8, G)
    mask = pl.pallas_call(
        functools.partial(_scores_kernel, bt=BT, g_tot=G, topk=topk),
        grid=(B, T // BT),
        in_specs=[
            pl.BlockSpec((1, BT, E), lambda b, tb: (b, tb, 0)),
            pl.BlockSpec((E, HD), lambda b, tb: (0, 0)),
            pl.BlockSpec((1, G, HD), lambda b, tb: (b, 0, 0)),
        ],
        out_specs=pl.BlockSpec((1, BT, G), lambda b, tb: (b, tb, 0)),
        out_shape=jax.ShapeDtypeStruct((B, T, G), jnp.bool_),
    )(x, wq, keys)
    group_ends = jnp.minimum(jnp.arange(R - 1, G * R, R), T - 1)
    return mask, group_ends


# speedup vs baseline: 1.0837x; 1.0315x over previous
import functools
import math

import jax
import jax.numpy as jnp
from jax.experimental import pallas as pl
from jax.experimental.pallas import tpu as pltpu

H = 16
D = 64
HD = H * D
R = 4
EPS = 1e-6
F32 = jnp.float32
BF16 = jnp.bfloat16
HIGHEST = jax.lax.Precision.HIGHEST


def _head_pool():
    i = jax.lax.broadcasted_iota(jnp.int32, (HD, H), 0) // D
    j = jax.lax.broadcasted_iota(jnp.int32, (HD, H), 1)
    return (i == j).astype(F32)


def _head_bcast():
    i = jax.lax.broadcasted_iota(jnp.int32, (H, HD), 0)
    j = jax.lax.broadcasted_iota(jnp.int32, (H, HD), 1) // D
    return (i == j).astype(F32)


def _rms_scale(v):
    ssq = jnp.dot(v * v, _head_pool(), preferred_element_type=F32,
                  precision=HIGHEST)
    ssqb = jnp.dot(ssq, _head_bcast(), preferred_element_type=F32,
                   precision=HIGHEST)
    return jax.lax.rsqrt(ssqb * (1.0 / D) + EPS)


def _proj_kernel(x_ref, wq_ref, wk_ref, wg_ref, ape_ref,
                 qn_ref, keys_ref, *, bg):
    x = x_ref[0].astype(BF16)
    q = jnp.dot(x, wq_ref[...], preferred_element_type=F32)
    qn_ref[0] = (q * _rms_scale(q)).astype(BF16)
    k = jnp.dot(x, wk_ref[...], preferred_element_type=F32)
    g = jnp.dot(x, wg_ref[...], preferred_element_type=F32)
    g = g + ape_ref[...]
    g = g - jnp.max(g)
    e = jnp.exp(g)
    rows = R * bg
    seg = (jax.lax.broadcasted_iota(jnp.int32, (bg, rows), 0)
           == jax.lax.broadcasted_iota(jnp.int32, (bg, rows), 1) // R)
    segf = seg.astype(F32)
    both = jnp.concatenate([e, e * k], axis=1)
    cmp = jnp.dot(segf, both, preferred_element_type=F32, precision=HIGHEST)
    keys = cmp[:, HD:] / cmp[:, :HD]
    keys_ref[0] = (keys * _rms_scale(keys)).astype(BF16)


def _scores_kernel(qn_ref, keys_ref, out_ref, *, bt, g_tot, topk):
    tb = pl.program_id(1)
    s = jax.lax.dot_general(qn_ref[0], keys_ref[0], (((1,), (1,)), ((), ())),
                            preferred_element_type=F32)
    t_idx = tb * bt + jax.lax.broadcasted_iota(jnp.int32, (bt, g_tot), 0)
    g_end = R * jax.lax.broadcasted_iota(jnp.int32, (bt, g_tot), 1) + (R - 1)
    causal = g_end <= t_idx
    neg = jnp.float32(-jnp.inf)
    s = jnp.where(causal, s, neg)
    cur = s
    th = None
    for _ in range(topk):
        th = jnp.max(cur, axis=1, keepdims=True)
        cur = jnp.where(cur >= th, neg, cur)
    out_ref[0] = causal & (s >= th)


def kernel(x, Wq, Wk, Wg, ape):
    B, T, E = x.shape
    G = T // R
    BG = min(64, G)
    BT = min(512, T)
    wq = Wq.astype(BF16)
    wk = Wk.astype(BF16)
    wg = Wg.astype(BF16)
    ape2 = jnp.tile(ape.reshape(R, HD), (BG, 1))
    qn, keys = pl.pallas_call(
        functools.partial(_proj_kernel, bg=BG),
        grid=(B, G // BG),
        in_specs=[
            pl.BlockSpec((1, R * BG, E), lambda b, gb: (b, gb, 0)),
            pl.BlockSpec((E, HD), lambda b, gb: (0, 0)),
            pl.BlockSpec((E, HD), lambda b, gb: (0, 0)),
            pl.BlockSpec((E, HD), lambda b, gb: (0, 0)),
            pl.BlockSpec((R * BG, HD), lambda b, gb: (0, 0)),
        ],
        out_specs=[
            pl.BlockSpec((1, R * BG, HD), lambda b, gb: (b, gb, 0)),
            pl.BlockSpec((1, BG, HD), lambda b, gb: (b, gb, 0)),
        ],
        out_shape=[
            jax.ShapeDtypeStruct((B, T, HD), BF16),
            jax.ShapeDtypeStruct((B, G, HD), BF16),
        ],
        compiler_params=pltpu.CompilerParams(
            dimension_semantics=("parallel", "parallel")),
    )(x, wq, wk, wg, ape2)
    topk = min(8, G)
    mask = pl.pallas_call(
        functools.partial(_scores_kernel, bt=BT, g_tot=G, topk=topk),
        grid=(B, T // BT),
        in_specs=[
            pl.BlockSpec((1, BT, HD), lambda b, tb: (b, tb, 0)),
            pl.BlockSpec((1, G, HD), lambda b, tb: (b, 0, 0)),
        ],
        out_specs=pl.BlockSpec((1, BT, G), lambda b, tb: (b, tb, 0)),
        out_shape=jax.ShapeDtypeStruct((B, T, G), jnp.bool_),
        compiler_params=pltpu.CompilerParams(
            dimension_semantics=("parallel", "parallel")),
    )(qn, keys)
    group_ends = jnp.minimum(jnp.arange(R - 1, G * R, R), T - 1)
    return mask, group_ends


# speedup vs baseline: 1.1294x; 1.0422x over previous
import functools
import math

import jax
import jax.numpy as jnp
from jax.experimental import pallas as pl
from jax.experimental.pallas import tpu as pltpu

H = 16
D = 64
HD = H * D
R = 4
EPS = 1e-6
F32 = jnp.float32
BF16 = jnp.bfloat16
HIGHEST = jax.lax.Precision.HIGHEST


def _head_pool():
    i = jax.lax.broadcasted_iota(jnp.int32, (HD, H), 0) // D
    j = jax.lax.broadcasted_iota(jnp.int32, (HD, H), 1)
    return (i == j).astype(F32)


def _head_bcast():
    i = jax.lax.broadcasted_iota(jnp.int32, (H, HD), 0)
    j = jax.lax.broadcasted_iota(jnp.int32, (H, HD), 1) // D
    return (i == j).astype(F32)


def _rms_scale(v):
    ssq = jnp.dot(v * v, _head_pool(), preferred_element_type=F32,
                  precision=HIGHEST)
    ssqb = jnp.dot(ssq, _head_bcast(), preferred_element_type=F32,
                   precision=HIGHEST)
    return jax.lax.rsqrt(ssqb * (1.0 / D) + EPS)


def _proj_kernel(x_ref, wq_ref, wk_ref, wg_ref, ape_ref,
                 qn_ref, keys_ref, *, bg):
    x = x_ref[0].astype(BF16)
    q = jnp.dot(x, wq_ref[...], preferred_element_type=F32)
    qn_ref[0] = (q * _rms_scale(q)).astype(BF16)
    k = jnp.dot(x, wk_ref[...], preferred_element_type=F32)
    g = jnp.dot(x, wg_ref[...], preferred_element_type=F32)
    g = g + ape_ref[...]
    g = g - jnp.max(g)
    e = jnp.exp(g)
    sub = 64
    srows = R * sub
    seg = (jax.lax.broadcasted_iota(jnp.int32, (sub, srows), 0)
           == jax.lax.broadcasted_iota(jnp.int32, (sub, srows), 1) // R)
    segf = seg.astype(F32)
    both = jnp.concatenate([e, e * k], axis=1)
    parts = []
    for c in range(bg // sub):
        cmp = jnp.dot(segf, both[c * srows:(c + 1) * srows, :],
                      preferred_element_type=F32, precision=HIGHEST)
        parts.append(cmp[:, HD:] / cmp[:, :HD])
    keys = jnp.concatenate(parts, axis=0) if len(parts) > 1 else parts[0]
    keys_ref[0] = (keys * _rms_scale(keys)).astype(BF16)


def _scores_kernel(qn_ref, keys_ref, out_ref, *, bt, g_tot, topk):
    tb = pl.program_id(1)
    s = jax.lax.dot_general(qn_ref[0], keys_ref[0], (((1,), (1,)), ((), ())),
                            preferred_element_type=F32)
    t_idx = tb * bt + jax.lax.broadcasted_iota(jnp.int32, (bt, g_tot), 0)
    g_end = R * jax.lax.broadcasted_iota(jnp.int32, (bt, g_tot), 1) + (R - 1)
    causal = g_end <= t_idx
    neg = jnp.float32(-jnp.inf)
    s = jnp.where(causal, s, neg)
    cur = s
    th = None
    for _ in range(topk):
        th = jnp.max(cur, axis=1, keepdims=True)
        cur = jnp.where(cur >= th, neg, cur)
    out_ref[0] = causal & (s >= th)


def kernel(x, Wq, Wk, Wg, ape):
    B, T, E = x.shape
    G = T // R
    BG = min(128, G)
    BT = min(1024, T)
    wq = Wq.astype(BF16)
    wk = Wk.astype(BF16)
    wg = Wg.astype(BF16)
    ape2 = jnp.tile(ape.reshape(R, HD), (BG, 1))
    qn, keys = pl.pallas_call(
        functools.partial(_proj_kernel, bg=BG),
        grid=(B, G // BG),
        in_specs=[
            pl.BlockSpec((1, R * BG, E), lambda b, gb: (b, gb, 0)),
            pl.BlockSpec((E, HD), lambda b, gb: (0, 0)),
            pl.BlockSpec((E, HD), lambda b, gb: (0, 0)),
            pl.BlockSpec((E, HD), lambda b, gb: (0, 0)),
            pl.BlockSpec((R * BG, HD), lambda b, gb: (0, 0)),
        ],
        out_specs=[
            pl.BlockSpec((1, R * BG, HD), lambda b, gb: (b, gb, 0)),
            pl.BlockSpec((1, BG, HD), lambda b, gb: (b, gb, 0)),
        ],
        out_shape=[
            jax.ShapeDtypeStruct((B, T, HD), BF16),
            jax.ShapeDtypeStruct((B, G, HD), BF16),
        ],
        compiler_params=pltpu.CompilerParams(
            dimension_semantics=("parallel", "parallel")),
    )(x, wq, wk, wg, ape2)
    topk = min(8, G)
    mask = pl.pallas_call(
        functools.partial(_scores_kernel, bt=BT, g_tot=G, topk=topk),
        grid=(B, T // BT),
        in_specs=[
            pl.BlockSpec((1, BT, HD), lambda b, tb: (b, tb, 0)),
            pl.BlockSpec((1, G, HD), lambda b, tb: (b, 0, 0)),
        ],
        out_specs=pl.BlockSpec((1, BT, G), lambda b, tb: (b, tb, 0)),
        out_shape=jax.ShapeDtypeStruct((B, T, G), jnp.bool_),
        compiler_params=pltpu.CompilerParams(
            dimension_semantics=("parallel", "parallel")),
    )(qn, keys)
    group_ends = jnp.minimum(jnp.arange(R - 1, G * R, R), T - 1)
    return mask, group_ends
